# Initial kernel scaffold; baseline (speedup 1.0000x reference)
import jax
import jax.numpy as jnp
from jax.experimental import pallas as pl


def kernel(pos, W1a, b1a, W1b, b1b, W2a, b2a, W2b, b2b, Wc1, bc1, Wc2, bc2, Wl1, bl1, Wl2, bl2, Wl3, bl3, batch):
    raise NotImplementedError("write your pallas kernel here")


# speedup vs baseline: 3.8927x; 3.8927x over previous
import jax
import jax.numpy as jnp
from jax.experimental import pallas as pl
from jax.experimental.pallas import tpu as pltpu

_P = 512
_K = 16
_NC = 40


def _split(x):
    hi = x.astype(jnp.bfloat16)
    lo = (x - hi.astype(jnp.float32)).astype(jnp.bfloat16)
    return hi, lo


def _dot_nn(a, b):
    return jax.lax.dot_general(a, b, (((1,), (0,)), ((), ())),
                               preferred_element_type=jnp.float32)


def _dot_nt(a, b):
    return jax.lax.dot_general(a, b, (((1,), (1,)), ((), ())),
                               preferred_element_type=jnp.float32)


def _mm3(a, b):
    ah, al = _split(a)
    bh, bl = _split(b)
    return _dot_nn(ah, bh) + (_dot_nn(ah, bl) + _dot_nn(al, bh))


def _mm3_nt(a, b):
    ah, al = _split(a)
    bh, bl = _split(b)
    return _dot_nt(ah, bh) + (_dot_nt(ah, bl) + _dot_nt(al, bh))


def _net_kernel(pos_ref, W1a_ref, b1a_ref, W1b_ref, b1b_ref,
                W2a_ref, b2a_ref, W2b_ref, b2b_ref,
                Wc1_ref, bc1_ref, Wc2_ref, bc2_ref,
                Wl1_ref, bl1_ref, Wl2_ref, bl2_ref, Wl3_ref, bl3_ref,
                out_ref, oh_ref):
    pos = pos_ref[...]

    sq = jnp.sum(pos * pos, axis=1, keepdims=True)
    dm = sq + jnp.transpose(sq) - 2.0 * _mm3_nt(pos, pos)
    iota = jax.lax.broadcasted_iota(jnp.int32, (_P, _P), 1)

    def topk_body(k, dcur):
        minv = jnp.min(dcur, axis=1, keepdims=True)
        cand = jnp.where(dcur == minv, iota, _P)
        mini = jnp.min(cand, axis=1, keepdims=True)
        oh = iota == mini
        oh_ref[k] = oh.astype(jnp.bfloat16)
        return jnp.where(oh, jnp.float32(3.0e38), dcur)

    jax.lax.fori_loop(0, _K, topk_body, dm)

    def edge_layer(u, v, ba, wb, bb):
        uh, ul = _split(u)
        wbh, wbl = _split(wb)

        def body(k, acc):
            ohk = oh_ref[k]
            g = _dot_nn(ohk, uh) + _dot_nn(ohk, ul)
            m = jnp.maximum(g - v + ba, 0.0)
            mh, ml = _split(m)
            t = _dot_nn(mh, wbh) + (_dot_nn(mh, wbl) + _dot_nn(ml, wbh))
            return jnp.maximum(acc, t)

        acc = jax.lax.fori_loop(0, _K, body,
                                jnp.full((_P, 128), -jnp.inf, jnp.float32))
        return jnp.maximum(acc + bb, 0.0)

    w1a = W1a_ref[...]
    v1 = _mm3(pos, w1a[3:6])
    u1 = _mm3(pos, w1a[0:3] + w1a[3:6])
    h1 = edge_layer(u1, v1, b1a_ref[...], W1b_ref[...], b1b_ref[...])

    w2a = W2a_ref[...]
    pp = _mm3(pos, w2a[128:131])
    u2 = _mm3(h1, w2a[0:128]) + pp
    h2 = edge_layer(u2, pp, b2a_ref[...], W2b_ref[...], b2b_ref[...])

    def cheb(h, wc, bc):
        sqh = jnp.sum(h * h, axis=1, keepdims=True)
        dd = sqh + jnp.transpose(sqh) - 2.0 * _mm3_nt(h, h)
        adj = jnp.exp(-dd)
        dinv = 1.0 / jnp.sqrt(jnp.sum(adj, axis=1, keepdims=True))
        an = adj * dinv * jnp.transpose(dinv)
        x0 = h
        t = _mm3(x0, wc[0])
        x1 = x0 - _mm3(an, x0)
        t = t + _mm3(x1, wc[1])
        x2 = 2.0 * (x1 - _mm3(an, x1)) - x0
        t = t + _mm3(x2, wc[2])
        return jnp.maximum(t + bc, 0.0)

    h3 = cheb(h2, Wc1_ref[...], bc1_ref[...])
    h4 = cheb(h3, Wc2_ref[...], bc2_ref[...])

    g = jnp.max(h4, axis=0, keepdims=True)
    o = jnp.maximum(_mm3(g, Wl1_ref[...]) + bl1_ref[...], 0.0)
    o = jnp.maximum(_mm3(o, Wl2_ref[...]) + bl2_ref[...], 0.0)
    o = _mm3(o, Wl3_ref[...]) + bl3_ref[...]
    out_ref[0] = o


def kernel(pos, W1a, b1a, W1b, b1b, W2a, b2a, W2b, b2b, Wc1, bc1, Wc2, bc2,
           Wl1, bl1, Wl2, bl2, Wl3, bl3, batch):
    n = pos.shape[0]
    bn = n // _P
    full = lambda shape: pl.BlockSpec(shape, lambda b: (0,) * len(shape))
    row = lambda f: pl.BlockSpec((1, f), lambda b: (0, 0))

    out = pl.pallas_call(
        _net_kernel,
        grid=(bn,),
        in_specs=[
            pl.BlockSpec((_P, 3), lambda b: (b, 0)),
            full((6, 128)), row(128),
            full((128, 128)), row(128),
            full((131, 128)), row(128),
            full((128, 128)), row(128),
            full((3, 128, 512)), row(512),
            full((3, 512, 1024)), row(1024),
            full((1024, 512)), row(512),
            full((512, 128)), row(128),
            full((128, _NC)), row(_NC),
        ],
        out_specs=pl.BlockSpec((1, 1, _NC), lambda b: (b, 0, 0)),
        out_shape=jax.ShapeDtypeStruct((bn, 1, _NC), jnp.float32),
        scratch_shapes=[pltpu.VMEM((_K, _P, _P), jnp.bfloat16)],
    )(pos, W1a, b1a.reshape(1, 128), W1b, b1b.reshape(1, 128),
      W2a, b2a.reshape(1, 128), W2b, b2b.reshape(1, 128),
      Wc1, bc1.reshape(1, 512), Wc2, bc2.reshape(1, 1024),
      Wl1, bl1.reshape(1, 512), Wl2, bl2.reshape(1, 128),
      Wl3, bl3.reshape(1, _NC))
    return out.reshape(bn, _NC)


# speedup vs baseline: 5.8101x; 1.4926x over previous
import jax
import jax.numpy as jnp
from jax.experimental import pallas as pl
from jax.experimental.pallas import tpu as pltpu

_P = 512
_K = 16
_NC = 40
_BIG = 3.0e38


def _split(x):
    hi = x.astype(jnp.bfloat16)
    lo = (x - hi.astype(jnp.float32)).astype(jnp.bfloat16)
    return hi, lo


def _dot_nn(a, b):
    return jax.lax.dot_general(a, b, (((1,), (0,)), ((), ())),
                               preferred_element_type=jnp.float32)


def _dot_nt(a, b):
    return jax.lax.dot_general(a, b, (((1,), (1,)), ((), ())),
                               preferred_element_type=jnp.float32)


def _mm3(a, b):
    ah, al = _split(a)
    bh, bl = _split(b)
    return _dot_nn(ah, bh) + (_dot_nn(ah, bl) + _dot_nn(al, bh))


def _mm3_nt(a, b):
    ah, al = _split(a)
    bh, bl = _split(b)
    return _dot_nt(ah, bh) + (_dot_nt(ah, bl) + _dot_nt(al, bh))


def _net_kernel(pos_ref, W1a_ref, b1a_ref, WB1_ref, b1b_ref,
                W2a_ref, b2a_ref, WB2_ref, b2b_ref,
                WC1P_ref, bc1_ref, Wc2h_ref, bc2_ref,
                Wl1_ref, bl1_ref, Wl2_ref, bl2_ref, Wl3_ref, bl3_ref,
                out_ref, oh_ref, g_ref):
    bn = pl.num_programs(0)
    b = pl.program_id(0)
    pos = pos_ref[...]

    sq = jnp.sum(pos * pos, axis=1, keepdims=True)
    dm = sq + jnp.transpose(sq) - 2.0 * _mm3_nt(pos, pos)

    def topk_body(k, dcur):
        minv = jnp.min(dcur, axis=1, keepdims=True)
        oh = dcur == minv
        oh_ref[k] = oh.astype(jnp.bfloat16)
        return jnp.where(oh, _BIG, dcur)

    jax.lax.fori_loop(0, _K, topk_body, dm)

    def edge_layer(u, v, ba, wblk, bb):
        uh, ul = _split(u)
        uhl = jnp.concatenate([uh, ul], axis=1)

        def body(k, acc):
            ohk = oh_ref[k]
            gg = _dot_nn(ohk, uhl)
            g = gg[:, :128] + gg[:, 128:]
            m = jnp.maximum(g - v + ba, 0.0)
            mh, ml = _split(m)
            mhl = jnp.concatenate([mh, ml], axis=1)
            tt = _dot_nn(mhl, wblk)
            t = tt[:, :128] + tt[:, 128:]
            return jnp.maximum(acc, t)

        acc = jax.lax.fori_loop(0, _K, body,
                                jnp.full((_P, 128), -jnp.inf, jnp.float32))
        return jnp.maximum(acc + bb, 0.0)

    w1a = W1a_ref[...]
    v1 = _mm3(pos, w1a[3:6])
    u1 = _mm3(pos, w1a[0:3] + w1a[3:6])
    h1 = edge_layer(u1, v1, b1a_ref[...], WB1_ref[...], b1b_ref[...])

    w2a = W2a_ref[...]
    pp = _mm3(pos, w2a[128:131])
    u2 = _mm3(h1, w2a[0:128]) + pp
    h2 = edge_layer(u2, pp, b2a_ref[...], WB2_ref[...], b2b_ref[...])

    def adjacency(h):
        sqh = jnp.sum(h * h, axis=1, keepdims=True)
        dd = sqh + jnp.transpose(sqh) - 2.0 * _mm3_nt(h, h)
        adj = jnp.exp(-dd)
        dinv = 1.0 / jnp.sqrt(jnp.sum(adj, axis=1, keepdims=True))
        return adj * dinv * jnp.transpose(dinv)

    an = adjacency(h2)
    anh, anl = _split(an)
    anhl = jnp.concatenate([anh, anl], axis=1)
    wc1p = WC1P_ref[...]

    def lx1(x):
        xh, xl = _split(x)
        xhl = jnp.concatenate([xh, xl], axis=1)
        r = _dot_nn(anhl, jnp.concatenate([xhl, xhl], axis=0))
        return r[:, :128] + r[:, 128:]

    def xw1(x):
        xh, xl = _split(x)
        xhl = jnp.concatenate([xh, xl], axis=1)
        return xhl

    x0 = h2
    x0hl = xw1(x0)
    t = _dot_nn(x0hl, wc1p[0])
    x1 = x0 - lx1(x0)
    x1hl = xw1(x1)
    t = t + _dot_nn(x1hl, wc1p[1])
    x2 = 2.0 * (x1 - lx1(x1)) - x0
    t = t + _dot_nn(xw1(x2), wc1p[2])
    t = (t[:, :512] + t[:, 512:]) + bc1_ref[...]
    h3 = jnp.maximum(t, 0.0)

    an2 = adjacency(h3)
    an2h = an2.astype(jnp.bfloat16)
    wc2h = Wc2h_ref[...]
    y0 = h3
    y0h = y0.astype(jnp.bfloat16)
    s = _dot_nn(y0h, wc2h[0])
    y1 = y0 - _dot_nn(an2h, y0h)
    y1h = y1.astype(jnp.bfloat16)
    s = s + _dot_nn(y1h, wc2h[1])
    y2 = 2.0 * (y1 - _dot_nn(an2h, y1h)) - y0
    s = s + _dot_nn(y2.astype(jnp.bfloat16), wc2h[2])
    h4 = jnp.maximum(s + bc2_ref[...], 0.0)

    g_ref[pl.ds(b, 1), :] = jnp.max(h4, axis=0, keepdims=True)

    @pl.when(b == bn - 1)
    def _head():
        g = g_ref[...]
        o = jnp.maximum(_mm3(g, Wl1_ref[...]) + bl1_ref[...], 0.0)
        o = jnp.maximum(_mm3(o, Wl2_ref[...]) + bl2_ref[...], 0.0)
        o = _mm3(o, Wl3_ref[...]) + bl3_ref[...]
        out_ref[...] = o


def kernel(pos, W1a, b1a, W1b, b1b, W2a, b2a, W2b, b2b, Wc1, bc1, Wc2, bc2,
           Wl1, bl1, Wl2, bl2, Wl3, bl3, batch):
    n = pos.shape[0]
    bn = n // _P
    f32 = jnp.float32
    bf16 = jnp.bfloat16

    def wsplit(w):
        hi = w.astype(bf16)
        lo = (w - hi.astype(f32)).astype(bf16)
        return hi, lo

    def edge_block(w):
        wh, wl = wsplit(w)
        top = jnp.concatenate([wh, wl], axis=1)
        bot = jnp.concatenate([wh, jnp.zeros_like(wh)], axis=1)
        return jnp.concatenate([top, bot], axis=0)

    def cheb1_pack(w):
        wh, wl = wsplit(w)
        wn = jnp.concatenate([wh, wl], axis=2)
        return jnp.concatenate([wn, wn], axis=1)

    WB1 = edge_block(W1b)
    WB2 = edge_block(W2b)
    WC1P = cheb1_pack(Wc1)
    Wc2h = Wc2.astype(bf16)

    full = lambda shape: pl.BlockSpec(shape, lambda b: (0,) * len(shape))
    row = lambda f: pl.BlockSpec((1, f), lambda b: (0, 0))

    out = pl.pallas_call(
        _net_kernel,
        grid=(bn,),
        in_specs=[
            pl.BlockSpec((_P, 3), lambda b: (b, 0)),
            full((6, 128)), row(128),
            full((256, 256)), row(128),
            full((131, 128)), row(128),
            full((256, 256)), row(128),
            full((3, 256, 1024)), row(512),
            full((3, 512, 1024)), row(1024),
            full((1024, 512)), row(512),
            full((512, 128)), row(128),
            full((128, _NC)), row(_NC),
        ],
        out_specs=pl.BlockSpec((bn, _NC), lambda b: (0, 0)),
        out_shape=jax.ShapeDtypeStruct((bn, _NC), jnp.float32),
        scratch_shapes=[pltpu.VMEM((_K, _P, _P), jnp.bfloat16),
                        pltpu.VMEM((bn, 1024), jnp.float32)],
    )(pos, W1a, b1a.reshape(1, 128), WB1, b1b.reshape(1, 128),
      W2a, b2a.reshape(1, 128), WB2, b2b.reshape(1, 128),
      WC1P, bc1.reshape(1, 512), Wc2h, bc2.reshape(1, 1024),
      Wl1, bl1.reshape(1, 512), Wl2, bl2.reshape(1, 128),
      Wl3, bl3.reshape(1, _NC))
    return out


# speedup vs baseline: 9.4830x; 1.6322x over previous
import jax
import jax.numpy as jnp
from jax.experimental import pallas as pl
from jax.experimental.pallas import tpu as pltpu

_P = 512
_K = 16
_NC = 40
_BIG = 3.0e38


def _split(x):
    hi = x.astype(jnp.bfloat16)
    lo = (x - hi.astype(jnp.float32)).astype(jnp.bfloat16)
    return hi, lo


def _dot_nn(a, b):
    return jax.lax.dot_general(a, b, (((1,), (0,)), ((), ())),
                               preferred_element_type=jnp.float32)


def _dot_nt(a, b):
    return jax.lax.dot_general(a, b, (((1,), (1,)), ((), ())),
                               preferred_element_type=jnp.float32)


def _mm3(a, b):
    ah, al = _split(a)
    bh, bl = _split(b)
    return _dot_nn(ah, bh) + (_dot_nn(ah, bl) + _dot_nn(al, bh))


def _mm3_nt(a, b):
    ah, al = _split(a)
    bh, bl = _split(b)
    return _dot_nt(ah, bh) + (_dot_nt(ah, bl) + _dot_nt(al, bh))


def _net_kernel(pos_ref, W1a_ref, b1a_ref, WB1_ref, b1b_ref,
                W2a_ref, b2a_ref, WB2_ref, b2b_ref,
                WC1P_ref, bc1_ref, Wc2h_ref, bc2_ref,
                Wl1_ref, bl1_ref, Wl2_ref, bl2_ref, Wl3_ref, bl3_ref,
                out_ref, oh_ref, g_ref):
    bn = pl.num_programs(0)
    b = pl.program_id(0)
    pos = pos_ref[...]

    sq = jnp.sum(pos * pos, axis=1, keepdims=True)
    dm = sq + jnp.transpose(sq) - 2.0 * _mm3_nt(pos, pos)
    diag = (jax.lax.broadcasted_iota(jnp.int32, (_P, _P), 0)
            == jax.lax.broadcasted_iota(jnp.int32, (_P, _P), 1))
    d = jnp.where(diag, _BIG, dm)
    for k in range(_K - 1):
        minv = jnp.min(d, axis=1, keepdims=True)
        oh = d == minv
        oh_ref[k] = oh.astype(jnp.bfloat16)
        if k < _K - 2:
            d = jnp.where(oh, _BIG, d)

    def edge_layer(u, v, ba, wblk, bb):
        uh, ul = _split(u)
        uhl = jnp.concatenate([uh, ul], axis=1)
        w = v - ba

        m0 = jnp.maximum(u - w, 0.0)
        m0h, m0l = _split(m0)
        tt0 = _dot_nn(jnp.concatenate([m0h, m0l], axis=1), wblk)
        acc = tt0[:, :128] + tt0[:, 128:]

        for lo, hi in ((0, 8), (8, _K - 1)):
            cnt = hi - lo
            ohc = oh_ref[lo:hi].reshape(cnt * _P, _P)
            gg = _dot_nn(ohc, uhl)
            g3 = gg.reshape(cnt, _P, 256)
            m3 = jnp.maximum(g3[:, :, :128] + g3[:, :, 128:] - w[None], 0.0)
            m3h, m3l = _split(m3)
            mhl = jnp.concatenate([m3h, m3l], axis=2).reshape(cnt * _P, 256)
            tt = _dot_nn(mhl, wblk)
            t = (tt[:, :128] + tt[:, 128:]).reshape(cnt, _P, 128)
            acc = jnp.maximum(acc, jnp.max(t, axis=0))
        return jnp.maximum(acc + bb, 0.0)

    w1a = W1a_ref[...]
    v1 = _mm3(pos, w1a[3:6])
    u1 = _mm3(pos, w1a[0:3] + w1a[3:6])
    h1 = edge_layer(u1, v1, b1a_ref[...], WB1_ref[...], b1b_ref[...])

    w2a = W2a_ref[...]
    pp = _mm3(pos, w2a[128:131])
    u2 = _mm3(h1, w2a[0:128]) + pp
    h2 = edge_layer(u2, pp, b2a_ref[...], WB2_ref[...], b2b_ref[...])

    def adjacency(h):
        sqh = jnp.sum(h * h, axis=1, keepdims=True)
        dd = sqh + jnp.transpose(sqh) - 2.0 * _mm3_nt(h, h)
        adj = jnp.exp(-dd)
        dinv = 1.0 / jnp.sqrt(jnp.sum(adj, axis=1, keepdims=True))
        return adj * dinv * jnp.transpose(dinv)

    an = adjacency(h2)
    anh, anl = _split(an)
    anhl = jnp.concatenate([anh, anl], axis=1)
    wc1p = WC1P_ref[...]

    def lx1(x):
        xh, xl = _split(x)
        xhl = jnp.concatenate([xh, xl], axis=1)
        r = _dot_nn(anhl, jnp.concatenate([xhl, xhl], axis=0))
        return r[:, :128] + r[:, 128:]

    def xw1(x):
        xh, xl = _split(x)
        xhl = jnp.concatenate([xh, xl], axis=1)
        return xhl

    x0 = h2
    x0hl = xw1(x0)
    t = _dot_nn(x0hl, wc1p[0])
    x1 = x0 - lx1(x0)
    x1hl = xw1(x1)
    t = t + _dot_nn(x1hl, wc1p[1])
    x2 = 2.0 * (x1 - lx1(x1)) - x0
    t = t + _dot_nn(xw1(x2), wc1p[2])
    t = (t[:, :512] + t[:, 512:]) + bc1_ref[...]
    h3 = jnp.maximum(t, 0.0)

    an2 = adjacency(h3)
    an2h = an2.astype(jnp.bfloat16)
    wc2h = Wc2h_ref[...]
    y0 = h3
    y0h = y0.astype(jnp.bfloat16)
    s = _dot_nn(y0h, wc2h[0])
    y1 = y0 - _dot_nn(an2h, y0h)
    y1h = y1.astype(jnp.bfloat16)
    s = s + _dot_nn(y1h, wc2h[1])
    y2 = 2.0 * (y1 - _dot_nn(an2h, y1h)) - y0
    s = s + _dot_nn(y2.astype(jnp.bfloat16), wc2h[2])
    h4 = jnp.maximum(s + bc2_ref[...], 0.0)

    g_ref[pl.ds(b, 1), :] = jnp.max(h4, axis=0, keepdims=True)

    @pl.when(b == bn - 1)
    def _head():
        g = g_ref[...]
        o = jnp.maximum(_mm3(g, Wl1_ref[...]) + bl1_ref[...], 0.0)
        o = jnp.maximum(_mm3(o, Wl2_ref[...]) + bl2_ref[...], 0.0)
        o = _mm3(o, Wl3_ref[...]) + bl3_ref[...]
        out_ref[...] = o


def kernel(pos, W1a, b1a, W1b, b1b, W2a, b2a, W2b, b2b, Wc1, bc1, Wc2, bc2,
           Wl1, bl1, Wl2, bl2, Wl3, bl3, batch):
    n = pos.shape[0]
    bn = n // _P
    f32 = jnp.float32
    bf16 = jnp.bfloat16

    def wsplit(w):
        hi = w.astype(bf16)
        lo = (w - hi.astype(f32)).astype(bf16)
        return hi, lo

    def edge_block(w):
        wh, wl = wsplit(w)
        top = jnp.concatenate([wh, wl], axis=1)
        bot = jnp.concatenate([wh, jnp.zeros_like(wh)], axis=1)
        return jnp.concatenate([top, bot], axis=0)

    def cheb1_pack(w):
        wh, wl = wsplit(w)
        wn = jnp.concatenate([wh, wl], axis=2)
        return jnp.concatenate([wn, wn], axis=1)

    WB1 = edge_block(W1b)
    WB2 = edge_block(W2b)
    WC1P = cheb1_pack(Wc1)
    Wc2h = Wc2.astype(bf16)

    full = lambda shape: pl.BlockSpec(shape, lambda b: (0,) * len(shape))
    row = lambda f: pl.BlockSpec((1, f), lambda b: (0, 0))

    out = pl.pallas_call(
        _net_kernel,
        grid=(bn,),
        in_specs=[
            pl.BlockSpec((_P, 3), lambda b: (b, 0)),
            full((6, 128)), row(128),
            full((256, 256)), row(128),
            full((131, 128)), row(128),
            full((256, 256)), row(128),
            full((3, 256, 1024)), row(512),
            full((3, 512, 1024)), row(1024),
            full((1024, 512)), row(512),
            full((512, 128)), row(128),
            full((128, _NC)), row(_NC),
        ],
        out_specs=pl.BlockSpec((bn, _NC), lambda b: (0, 0)),
        out_shape=jax.ShapeDtypeStruct((bn, _NC), jnp.float32),
        scratch_shapes=[pltpu.VMEM((_K - 1, _P, _P), jnp.bfloat16),
                        pltpu.VMEM((bn, 1024), jnp.float32)],
    )(pos, W1a, b1a.reshape(1, 128), WB1, b1b.reshape(1, 128),
      W2a, b2a.reshape(1, 128), WB2, b2b.reshape(1, 128),
      WC1P, bc1.reshape(1, 512), Wc2h, bc2.reshape(1, 1024),
      Wl1, bl1.reshape(1, 512), Wl2, bl2.reshape(1, 128),
      Wl3, bl3.reshape(1, _NC))
    return out


# speedup vs baseline: 9.5224x; 1.0041x over previous
import jax
import jax.numpy as jnp
from jax.experimental import pallas as pl
from jax.experimental.pallas import tpu as pltpu

_P = 512
_K = 16
_NC = 40
_BIG = 3.0e38


def _split(x):
    hi = x.astype(jnp.bfloat16)
    lo = (x - hi.astype(jnp.float32)).astype(jnp.bfloat16)
    return hi, lo


def _dot_nn(a, b):
    return jax.lax.dot_general(a, b, (((1,), (0,)), ((), ())),
                               preferred_element_type=jnp.float32)


def _dot_nt(a, b):
    return jax.lax.dot_general(a, b, (((1,), (1,)), ((), ())),
                               preferred_element_type=jnp.float32)


def _mm3(a, b):
    ah, al = _split(a)
    bh, bl = _split(b)
    return _dot_nn(ah, bh) + (_dot_nn(ah, bl) + _dot_nn(al, bh))


def _mm3_nt(a, b):
    ah, al = _split(a)
    bh, bl = _split(b)
    return _dot_nt(ah, bh) + (_dot_nt(ah, bl) + _dot_nt(al, bh))


def _net_kernel(pos_ref, W1a_ref, b1a_ref, WB1_ref, b1b_ref,
                W2a_ref, b2a_ref, WB2_ref, b2b_ref,
                WC1P_ref, bc1_ref, Wc2h_ref, bc2_ref,
                Wl1_ref, bl1_ref, Wl2_ref, bl2_ref, Wl3_ref, bl3_ref,
                out_ref, oh_ref, g_ref):
    bn = pl.num_programs(0)
    b = pl.program_id(0)
    pos = pos_ref[...]

    sq = jnp.sum(pos * pos, axis=1, keepdims=True)
    dm = sq + jnp.transpose(sq) - 2.0 * _mm3_nt(pos, pos)
    diag = (jax.lax.broadcasted_iota(jnp.int32, (_P, _P), 0)
            == jax.lax.broadcasted_iota(jnp.int32, (_P, _P), 1))
    d = jnp.where(diag, _BIG, dm)
    for k in range(_K - 1):
        minv = jnp.min(d, axis=1, keepdims=True)
        oh = d == minv
        oh_ref[k] = oh.astype(jnp.bfloat16)
        if k < _K - 2:
            d = jnp.where(oh, _BIG, d)

    def edge_layer(u, v, ba, wblk, bb):
        uh, ul = _split(u)
        uhl = jnp.concatenate([uh, ul], axis=1)
        w = v - ba

        m0 = jnp.maximum(u - w, 0.0)
        tt0 = _dot_nn(m0.astype(jnp.bfloat16), wblk)
        acc = tt0[:, :128] + tt0[:, 128:]

        for lo, hi in ((0, 8), (8, _K - 1)):
            cnt = hi - lo
            ohc = oh_ref[lo:hi].reshape(cnt * _P, _P)
            gg = _dot_nn(ohc, uhl)
            g3 = gg.reshape(cnt, _P, 256)
            m3 = jnp.maximum(g3[:, :, :128] + g3[:, :, 128:] - w[None], 0.0)
            mh = m3.astype(jnp.bfloat16).reshape(cnt * _P, 128)
            tt = _dot_nn(mh, wblk)
            t = (tt[:, :128] + tt[:, 128:]).reshape(cnt, _P, 128)
            acc = jnp.maximum(acc, jnp.max(t, axis=0))
        return jnp.maximum(acc + bb, 0.0)

    w1a = W1a_ref[...]
    v1 = _mm3(pos, w1a[3:6])
    u1 = _mm3(pos, w1a[0:3] + w1a[3:6])
    h1 = edge_layer(u1, v1, b1a_ref[...], WB1_ref[...], b1b_ref[...])

    w2a = W2a_ref[...]
    pp = _mm3(pos, w2a[128:131])
    u2 = _mm3(h1, w2a[0:128]) + pp
    h2 = edge_layer(u2, pp, b2a_ref[...], WB2_ref[...], b2b_ref[...])

    def adjacency(h):
        sqh = jnp.sum(h * h, axis=1, keepdims=True)
        dd = sqh + jnp.transpose(sqh) - 2.0 * _mm3_nt(h, h)
        adj = jnp.exp(-dd)
        dinv = 1.0 / jnp.sqrt(jnp.sum(adj, axis=1, keepdims=True))
        return adj * dinv * jnp.transpose(dinv)

    an = adjacency(h2)
    anh, anl = _split(an)
    anhl = jnp.concatenate([anh, anl], axis=1)
    wc1p = WC1P_ref[...]

    def lx1(x):
        xh, xl = _split(x)
        xhl = jnp.concatenate([xh, xl], axis=1)
        r = _dot_nn(anhl, jnp.concatenate([xhl, xhl], axis=0))
        return r[:, :128] + r[:, 128:]

    def xw1(x):
        xh, xl = _split(x)
        xhl = jnp.concatenate([xh, xl], axis=1)
        return xhl

    x0 = h2
    x0hl = xw1(x0)
    t = _dot_nn(x0hl, wc1p[0])
    x1 = x0 - lx1(x0)
    x1hl = xw1(x1)
    t = t + _dot_nn(x1hl, wc1p[1])
    x2 = 2.0 * (x1 - lx1(x1)) - x0
    t = t + _dot_nn(xw1(x2), wc1p[2])
    t = (t[:, :512] + t[:, 512:]) + bc1_ref[...]
    h3 = jnp.maximum(t, 0.0)

    an2 = adjacency(h3)
    an2h = an2.astype(jnp.bfloat16)
    wc2h = Wc2h_ref[...]
    y0 = h3
    y0h = y0.astype(jnp.bfloat16)
    s = _dot_nn(y0h, wc2h[0])
    y1 = y0 - _dot_nn(an2h, y0h)
    y1h = y1.astype(jnp.bfloat16)
    s = s + _dot_nn(y1h, wc2h[1])
    y2 = 2.0 * (y1 - _dot_nn(an2h, y1h)) - y0
    s = s + _dot_nn(y2.astype(jnp.bfloat16), wc2h[2])
    h4 = jnp.maximum(s + bc2_ref[...], 0.0)

    g_ref[pl.ds(b, 1), :] = jnp.max(h4, axis=0, keepdims=True)

    @pl.when(b == bn - 1)
    def _head():
        g = g_ref[...]
        o = jnp.maximum(_mm3(g, Wl1_ref[...]) + bl1_ref[...], 0.0)
        o = jnp.maximum(_mm3(o, Wl2_ref[...]) + bl2_ref[...], 0.0)
        o = _mm3(o, Wl3_ref[...]) + bl3_ref[...]
        out_ref[...] = o


def kernel(pos, W1a, b1a, W1b, b1b, W2a, b2a, W2b, b2b, Wc1, bc1, Wc2, bc2,
           Wl1, bl1, Wl2, bl2, Wl3, bl3, batch):
    n = pos.shape[0]
    bn = n // _P
    f32 = jnp.float32
    bf16 = jnp.bfloat16

    def wsplit(w):
        hi = w.astype(bf16)
        lo = (w - hi.astype(f32)).astype(bf16)
        return hi, lo

    def edge_block(w):
        wh, wl = wsplit(w)
        return jnp.concatenate([wh, wl], axis=1)

    def cheb1_pack(w):
        wh, wl = wsplit(w)
        wn = jnp.concatenate([wh, wl], axis=2)
        return jnp.concatenate([wn, wn], axis=1)

    WB1 = edge_block(W1b)
    WB2 = edge_block(W2b)
    WC1P = cheb1_pack(Wc1)
    Wc2h = Wc2.astype(bf16)

    full = lambda shape: pl.BlockSpec(shape, lambda b: (0,) * len(shape))
    row = lambda f: pl.BlockSpec((1, f), lambda b: (0, 0))

    out = pl.pallas_call(
        _net_kernel,
        grid=(bn,),
        in_specs=[
            pl.BlockSpec((_P, 3), lambda b: (b, 0)),
            full((6, 128)), row(128),
            full((128, 256)), row(128),
            full((131, 128)), row(128),
            full((128, 256)), row(128),
            full((3, 256, 1024)), row(512),
            full((3, 512, 1024)), row(1024),
            full((1024, 512)), row(512),
            full((512, 128)), row(128),
            full((128, _NC)), row(_NC),
        ],
        out_specs=pl.BlockSpec((bn, _NC), lambda b: (0, 0)),
        out_shape=jax.ShapeDtypeStruct((bn, _NC), jnp.float32),
        scratch_shapes=[pltpu.VMEM((_K - 1, _P, _P), jnp.bfloat16),
                        pltpu.VMEM((bn, 1024), jnp.float32)],
    )(pos, W1a, b1a.reshape(1, 128), WB1, b1b.reshape(1, 128),
      W2a, b2a.reshape(1, 128), WB2, b2b.reshape(1, 128),
      WC1P, bc1.reshape(1, 512), Wc2h, bc2.reshape(1, 1024),
      Wl1, bl1.reshape(1, 512), Wl2, bl2.reshape(1, 128),
      Wl3, bl3.reshape(1, _NC))
    return out


# speedup vs baseline: 11.6798x; 1.2266x over previous
import jax
import jax.numpy as jnp
from jax.experimental import pallas as pl
from jax.experimental.pallas import tpu as pltpu

_P = 512
_K = 16
_NC = 40
_BIG = 3.0e38


def _split(x):
    hi = x.astype(jnp.bfloat16)
    lo = (x - hi.astype(jnp.float32)).astype(jnp.bfloat16)
    return hi, lo


def _dot_nn(a, b):
    return jax.lax.dot_general(a, b, (((1,), (0,)), ((), ())),
                               preferred_element_type=jnp.float32)


def _dot_nt(a, b):
    return jax.lax.dot_general(a, b, (((1,), (1,)), ((), ())),
                               preferred_element_type=jnp.float32)


def _mm3(a, b):
    ah, al = _split(a)
    bh, bl = _split(b)
    return _dot_nn(ah, bh) + (_dot_nn(ah, bl) + _dot_nn(al, bh))


def _mm3_nt(a, b):
    ah, al = _split(a)
    bh, bl = _split(b)
    return _dot_nt(ah, bh) + (_dot_nt(ah, bl) + _dot_nt(al, bh))


def _net_kernel(pos_ref, W1sT_ref, W1bT_ref, b1aT_ref, WB1T_ref, b1bT_ref,
                W2hT_ref, W2pT_ref, b2aT_ref, WB2T_ref, b2bT_ref,
                WC1P_ref, bc1_ref, Wc2h_ref, bc2_ref,
                Wl1_ref, bl1_ref, Wl2_ref, bl2_ref, Wl3_ref, bl3_ref,
                out_ref, oh_ref, g_ref):
    bn = pl.num_programs(0)
    b = pl.program_id(0)
    pos = pos_ref[...]
    posT = jnp.transpose(pos)

    sqr = jnp.sum(posT * posT, axis=0, keepdims=True)
    sqc = jnp.transpose(sqr)
    dm = sqc + sqr - 2.0 * _mm3_nt(pos, pos)
    diag = (jax.lax.broadcasted_iota(jnp.int32, (_P, _P), 0)
            == jax.lax.broadcasted_iota(jnp.int32, (_P, _P), 1))
    d = jnp.where(diag, _BIG, dm)
    for k in range(_K - 1):
        minv = jnp.min(d, axis=0, keepdims=True)
        oh = d == minv
        oh_ref[:, k * _P:(k + 1) * _P] = oh.astype(jnp.bfloat16)
        if k < _K - 2:
            d = jnp.where(oh, _BIG, d)

    ohstack = oh_ref[...]

    def edge_layer(uT, vT, baT, wbTh, wbTl, bbT):
        uTh, uTl = _split(uT)
        wT = vT - baT

        m0 = jnp.maximum(uT - wT, 0.0).astype(jnp.bfloat16)
        accT = _dot_nn(wbTh, m0) + _dot_nn(wbTl, m0)

        gall = _dot_nn(uTh, ohstack) + _dot_nn(uTl, ohstack)
        wT5 = jnp.concatenate([wT] * 5, axis=1)
        for c in range(3):
            gc = gall[:, c * 5 * _P:(c + 1) * 5 * _P]
            mc = jnp.maximum(gc - wT5, 0.0).astype(jnp.bfloat16)
            tc = _dot_nn(wbTh, mc) + _dot_nn(wbTl, mc)
            for k in range(5):
                accT = jnp.maximum(accT, tc[:, k * _P:(k + 1) * _P])
        return jnp.maximum(accT + bbT, 0.0)

    u1T = _mm3(W1sT_ref[...], posT)
    v1T = _mm3(W1bT_ref[...], posT)
    h1T = edge_layer(u1T, v1T, b1aT_ref[...],
                     WB1T_ref[0], WB1T_ref[1], b1bT_ref[...])

    ppT = _mm3(W2pT_ref[...], posT)
    u2T = _mm3(W2hT_ref[...], h1T) + ppT
    h2T = edge_layer(u2T, ppT, b2aT_ref[...],
                     WB2T_ref[0], WB2T_ref[1], b2bT_ref[...])

    h2 = jnp.transpose(h2T)

    def adjacency(h):
        sqh = jnp.sum(h * h, axis=1, keepdims=True)
        dd = sqh + jnp.transpose(sqh) - 2.0 * _mm3_nt(h, h)
        adj = jnp.exp(-dd)
        dinv = 1.0 / jnp.sqrt(jnp.sum(adj, axis=1, keepdims=True))
        return adj * dinv * jnp.transpose(dinv)

    an = adjacency(h2)
    anh, anl = _split(an)
    anhl = jnp.concatenate([anh, anl], axis=1)
    wc1p = WC1P_ref[...]

    def lx1(x):
        xh, xl = _split(x)
        xhl = jnp.concatenate([xh, xl], axis=1)
        r = _dot_nn(anhl, jnp.concatenate([xhl, xhl], axis=0))
        return r[:, :128] + r[:, 128:]

    def xw1(x):
        xh, xl = _split(x)
        return jnp.concatenate([xh, xl], axis=1)

    x0 = h2
    t = _dot_nn(xw1(x0), wc1p[0])
    x1 = x0 - lx1(x0)
    t = t + _dot_nn(xw1(x1), wc1p[1])
    x2 = 2.0 * (x1 - lx1(x1)) - x0
    t = t + _dot_nn(xw1(x2), wc1p[2])
    t = (t[:, :512] + t[:, 512:]) + bc1_ref[...]
    h3 = jnp.maximum(t, 0.0)

    an2 = adjacency(h3)
    an2h = an2.astype(jnp.bfloat16)
    wc2h = Wc2h_ref[...]
    y0 = h3
    y0h = y0.astype(jnp.bfloat16)
    s = _dot_nn(y0h, wc2h[0])
    y1 = y0 - _dot_nn(an2h, y0h)
    y1h = y1.astype(jnp.bfloat16)
    s = s + _dot_nn(y1h, wc2h[1])
    y2 = 2.0 * (y1 - _dot_nn(an2h, y1h)) - y0
    s = s + _dot_nn(y2.astype(jnp.bfloat16), wc2h[2])
    h4 = jnp.maximum(s + bc2_ref[...], 0.0)

    g_ref[pl.ds(b, 1), :] = jnp.max(h4, axis=0, keepdims=True)

    @pl.when(b == bn - 1)
    def _head():
        g = g_ref[...]
        o = jnp.maximum(_mm3(g, Wl1_ref[...]) + bl1_ref[...], 0.0)
        o = jnp.maximum(_mm3(o, Wl2_ref[...]) + bl2_ref[...], 0.0)
        o = _mm3(o, Wl3_ref[...]) + bl3_ref[...]
        out_ref[...] = o


def kernel(pos, W1a, b1a, W1b, b1b, W2a, b2a, W2b, b2b, Wc1, bc1, Wc2, bc2,
           Wl1, bl1, Wl2, bl2, Wl3, bl3, batch):
    n = pos.shape[0]
    bn = n // _P
    f32 = jnp.float32
    bf16 = jnp.bfloat16

    def wsplit(w):
        hi = w.astype(bf16)
        lo = (w - hi.astype(f32)).astype(bf16)
        return hi, lo

    def edge_blockT(w):
        wh, wl = wsplit(w.T)
        return jnp.stack([wh, wl], axis=0)

    def cheb1_pack(w):
        wh, wl = wsplit(w)
        wn = jnp.concatenate([wh, wl], axis=2)
        return jnp.concatenate([wn, wn], axis=1)

    W1sT = (W1a[0:3] + W1a[3:6]).T
    W1bT = W1a[3:6].T
    W2hT = W2a[0:128].T
    W2pT = W2a[128:131].T
    WB1T = edge_blockT(W1b)
    WB2T = edge_blockT(W2b)
    WC1P = cheb1_pack(Wc1)
    Wc2h = Wc2.astype(bf16)

    full = lambda shape: pl.BlockSpec(shape, lambda b: (0,) * len(shape))
    row = lambda f: pl.BlockSpec((1, f), lambda b: (0, 0))
    col = lambda f: pl.BlockSpec((f, 1), lambda b: (0, 0))

    out = pl.pallas_call(
        _net_kernel,
        grid=(bn,),
        in_specs=[
            pl.BlockSpec((_P, 3), lambda b: (b, 0)),
            full((128, 3)), full((128, 3)), col(128),
            full((2, 128, 128)), col(128),
            full((128, 128)), full((128, 3)), col(128),
            full((2, 128, 128)), col(128),
            full((3, 256, 1024)), row(512),
            full((3, 512, 1024)), row(1024),
            full((1024, 512)), row(512),
            full((512, 128)), row(128),
            full((128, _NC)), row(_NC),
        ],
        out_specs=pl.BlockSpec((bn, _NC), lambda b: (0, 0)),
        out_shape=jax.ShapeDtypeStruct((bn, _NC), jnp.float32),
        scratch_shapes=[pltpu.VMEM((_P, (_K - 1) * _P), jnp.bfloat16),
                        pltpu.VMEM((bn, 1024), jnp.float32)],
    )(pos, W1sT, W1bT, b1a.reshape(128, 1), WB1T, b1b.reshape(128, 1),
      W2hT, W2pT, b2a.reshape(128, 1), WB2T, b2b.reshape(128, 1),
      WC1P, bc1.reshape(1, 512), Wc2h, bc2.reshape(1, 1024),
      Wl1, bl1.reshape(1, 512), Wl2, bl2.reshape(1, 128),
      Wl3, bl3.reshape(1, _NC))
    return out


# speedup vs baseline: 14.1029x; 1.2075x over previous
import jax
import jax.numpy as jnp
from jax.experimental import pallas as pl
from jax.experimental.pallas import tpu as pltpu

_P = 512
_K = 16
_NC = 40
_BIG = 3.0e38


def _split(x):
    hi = x.astype(jnp.bfloat16)
    lo = (x - hi.astype(jnp.float32)).astype(jnp.bfloat16)
    return hi, lo


def _dot_nn(a, b):
    return jax.lax.dot_general(a, b, (((1,), (0,)), ((), ())),
                               preferred_element_type=jnp.float32)


def _dot_nt(a, b):
    return jax.lax.dot_general(a, b, (((1,), (1,)), ((), ())),
                               preferred_element_type=jnp.float32)


def _mm3(a, b):
    ah, al = _split(a)
    bh, bl = _split(b)
    return _dot_nn(ah, bh) + (_dot_nn(ah, bl) + _dot_nn(al, bh))


def _mm3_nt(a, b):
    ah, al = _split(a)
    bh, bl = _split(b)
    return _dot_nt(ah, bh) + (_dot_nt(ah, bl) + _dot_nt(al, bh))


def _sym2_nt(a):
    ah, al = _split(a)
    return _dot_nt(ah, ah) + _dot_nt(ah, al)


def _net_kernel(pos_ref, W1sT_ref, W1bT_ref, b1aT_ref, WB1T_ref, b1bT_ref,
                W2hT_ref, W2pT_ref, b2aT_ref, WB2T_ref, b2bT_ref,
                WC1P_ref, bc1_ref, Wc2h_ref, bc2_ref,
                Wl1_ref, bl1_ref, Wl2_ref, bl2_ref, Wl3_ref, bl3_ref,
                out_ref, oh_ref, g_ref):
    bn = pl.num_programs(0)
    b = pl.program_id(0)
    pos = pos_ref[...]
    posT = jnp.transpose(pos)

    sqr = jnp.sum(posT * posT, axis=0, keepdims=True)
    sqc = jnp.transpose(sqr)
    dm = sqc + sqr - 2.0 * _mm3_nt(pos, pos)
    diag = (jax.lax.broadcasted_iota(jnp.int32, (_P, _P), 0)
            == jax.lax.broadcasted_iota(jnp.int32, (_P, _P), 1))
    d = jnp.where(diag, _BIG, dm)
    for k in range(_K - 1):
        minv = jnp.min(d, axis=0, keepdims=True)
        oh = d == minv
        oh_ref[:, k * _P:(k + 1) * _P] = oh.astype(jnp.bfloat16)
        if k < _K - 2:
            d = jnp.where(oh, _BIG, d)

    ohstack = oh_ref[...]

    def edge_layer(uT, vT, baT, wbTh, bbT):
        uTh = uT.astype(jnp.bfloat16)
        wT = vT - baT

        m0 = jnp.maximum(uT - wT, 0.0).astype(jnp.bfloat16)
        accT = _dot_nn(wbTh, m0)

        gall = _dot_nn(uTh, ohstack)
        wT5 = jnp.concatenate([wT] * 5, axis=1)
        for c in range(3):
            gc = gall[:, c * 5 * _P:(c + 1) * 5 * _P]
            mc = jnp.maximum(gc - wT5, 0.0).astype(jnp.bfloat16)
            tc = _dot_nn(wbTh, mc)
            for k in range(5):
                accT = jnp.maximum(accT, tc[:, k * _P:(k + 1) * _P])
        return jnp.maximum(accT + bbT, 0.0)

    u1T = _mm3(W1sT_ref[...], posT)
    v1T = _mm3(W1bT_ref[...], posT)
    h1T = edge_layer(u1T, v1T, b1aT_ref[...], WB1T_ref[...], b1bT_ref[...])

    ppT = _mm3(W2pT_ref[...], posT)
    u2T = _mm3(W2hT_ref[...], h1T) + ppT
    h2T = edge_layer(u2T, ppT, b2aT_ref[...], WB2T_ref[...], b2bT_ref[...])

    h2 = jnp.transpose(h2T)

    def adjacency(h):
        sqh = jnp.sum(h * h, axis=1, keepdims=True)
        dd = sqh + jnp.transpose(sqh) - 2.0 * _sym2_nt(h)
        adj = jnp.exp(-dd)
        dinv = 1.0 / jnp.sqrt(jnp.sum(adj, axis=1, keepdims=True))
        return adj * dinv * jnp.transpose(dinv)

    an = adjacency(h2)
    anh = an.astype(jnp.bfloat16)
    wc1h = WC1P_ref[...]

    x0 = h2
    x0h = x0.astype(jnp.bfloat16)
    t = _dot_nn(x0h, wc1h[0])
    x1 = x0 - _dot_nn(anh, x0h)
    x1h = x1.astype(jnp.bfloat16)
    t = t + _dot_nn(x1h, wc1h[1])
    x2 = 2.0 * (x1 - _dot_nn(anh, x1h)) - x0
    t = t + _dot_nn(x2.astype(jnp.bfloat16), wc1h[2])
    h3 = jnp.maximum(t + bc1_ref[...], 0.0)

    an2 = adjacency(h3)
    an2h = an2.astype(jnp.bfloat16)
    wc2h = Wc2h_ref[...]
    y0 = h3
    y0h = y0.astype(jnp.bfloat16)
    s = _dot_nn(y0h, wc2h[0])
    y1 = y0 - _dot_nn(an2h, y0h)
    y1h = y1.astype(jnp.bfloat16)
    s = s + _dot_nn(y1h, wc2h[1])
    y2 = 2.0 * (y1 - _dot_nn(an2h, y1h)) - y0
    s = s + _dot_nn(y2.astype(jnp.bfloat16), wc2h[2])
    h4 = jnp.maximum(s + bc2_ref[...], 0.0)

    g_ref[pl.ds(b, 1), :] = jnp.max(h4, axis=0, keepdims=True)

    @pl.when(b == bn - 1)
    def _head():
        g = g_ref[...]
        o = jnp.maximum(_mm3(g, Wl1_ref[...]) + bl1_ref[...], 0.0)
        o = jnp.maximum(_mm3(o, Wl2_ref[...]) + bl2_ref[...], 0.0)
        o = _mm3(o, Wl3_ref[...]) + bl3_ref[...]
        out_ref[...] = o


def kernel(pos, W1a, b1a, W1b, b1b, W2a, b2a, W2b, b2b, Wc1, bc1, Wc2, bc2,
           Wl1, bl1, Wl2, bl2, Wl3, bl3, batch):
    n = pos.shape[0]
    bn = n // _P
    f32 = jnp.float32
    bf16 = jnp.bfloat16

    def wsplit(w):
        hi = w.astype(bf16)
        lo = (w - hi.astype(f32)).astype(bf16)
        return hi, lo

    def edge_blockT(w):
        return w.T.astype(bf16)

    def cheb1_pack(w):
        return w.astype(bf16)

    W1sT = (W1a[0:3] + W1a[3:6]).T
    W1bT = W1a[3:6].T
    W2hT = W2a[0:128].T
    W2pT = W2a[128:131].T
    WB1T = edge_blockT(W1b)
    WB2T = edge_blockT(W2b)
    WC1P = cheb1_pack(Wc1)
    Wc2h = Wc2.astype(bf16)

    full = lambda shape: pl.BlockSpec(shape, lambda b: (0,) * len(shape))
    row = lambda f: pl.BlockSpec((1, f), lambda b: (0, 0))
    col = lambda f: pl.BlockSpec((f, 1), lambda b: (0, 0))

    out = pl.pallas_call(
        _net_kernel,
        grid=(bn,),
        in_specs=[
            pl.BlockSpec((_P, 3), lambda b: (b, 0)),
            full((128, 3)), full((128, 3)), col(128),
            full((128, 128)), col(128),
            full((128, 128)), full((128, 3)), col(128),
            full((128, 128)), col(128),
            full((3, 128, 512)), row(512),
            full((3, 512, 1024)), row(1024),
            full((1024, 512)), row(512),
            full((512, 128)), row(128),
            full((128, _NC)), row(_NC),
        ],
        out_specs=pl.BlockSpec((bn, _NC), lambda b: (0, 0)),
        out_shape=jax.ShapeDtypeStruct((bn, _NC), jnp.float32),
        scratch_shapes=[pltpu.VMEM((_P, (_K - 1) * _P), jnp.bfloat16),
                        pltpu.VMEM((bn, 1024), jnp.float32)],
    )(pos, W1sT, W1bT, b1a.reshape(128, 1), WB1T, b1b.reshape(128, 1),
      W2hT, W2pT, b2a.reshape(128, 1), WB2T, b2b.reshape(128, 1),
      WC1P, bc1.reshape(1, 512), Wc2h, bc2.reshape(1, 1024),
      Wl1, bl1.reshape(1, 512), Wl2, bl2.reshape(1, 128),
      Wl3, bl3.reshape(1, _NC))
    return out
